# baseline (speedup 1.0000x reference)
import functools

import jax
import jax.numpy as jnp
from jax import lax
from jax.experimental import pallas as pl
from jax.experimental.pallas import tpu as pltpu
from jax.experimental.pallas import tpu_sc as plsc

_N = 32768
_LANES = 128
_CHUNKS = _N // _LANES


_L = 16
_NC = 2
_NS = 16
_NW = _NC * _NS
_SC_ROWS = 32
_TC_ROWS = 96
_TC_BLOCK = 16
_CHUNK = 8192
_NCHUNKS = _N // _CHUNK
_U = 8

_LN2 = 0.6931471805599453
_SQRT2 = 1.4142135623730951
_LOG2_COEF = (
    2.8978064198215137e-08,
    1.4426949525633355,
    -0.7213581907292554,
    0.4809198929627641,
    -0.3600796321545737,
    0.28720812481307534,
    -0.2504655854924836,
    0.23321562654833375,
    -0.14022713339628334,
)

_mesh = plsc.VectorSubcoreMesh(core_axis_name="c", subcore_axis_name="s")


def _keys_for(pv, uv):
    ub = lax.bitcast_convert_type(uv, jnp.int32)
    e = (ub >> 23) - 127
    m = lax.bitcast_convert_type((ub & 0x7FFFFF) | 0x3F800000, jnp.float32)
    adj = m > jnp.float32(_SQRT2)
    m = jnp.where(adj, m * jnp.float32(0.5), m)
    e = e + adj.astype(jnp.int32)
    t = m - jnp.float32(1.0)
    acc = jnp.full((_L,), _LOG2_COEF[-1], jnp.float32)
    for c in _LOG2_COEF[-2::-1]:
        acc = acc * t + jnp.float32(c)
    logu = (e.astype(jnp.float32) + acc) * jnp.float32(_LN2)
    inner = jnp.maximum(-logu, jnp.float32(1e-9))
    r = (pv + jnp.float32(1e-5)) / inner
    return lax.bitcast_convert_type(r, jnp.int32)


def _digit_select(hist_ref, nbins, need):
    def body(v, carry):
        run, dcnt, cbel = carry
        h = hist_ref[pl.ds(v * _L, _L)]
        s = plsc.cumsum(h) + run
        mlt = s < need
        dcnt = dcnt + jnp.sum(mlt.astype(jnp.int32))
        cbel = cbel + jnp.sum(jnp.where(mlt, h, 0))
        run = run + jnp.sum(h)
        return run, dcnt, cbel

    z = jnp.int32(0)
    _, D, c_below = lax.fori_loop(0, nbins // _L, body, (z, z, z))
    return D, c_below


def _zero_hist(hist_ref, nbins):
    def body(v, _):
        hist_ref[pl.ds(v * _L, _L)] = jnp.zeros((_L,), jnp.int32)
        return 0

    lax.fori_loop(0, nbins // _L, body, 0)


@functools.partial(
    pl.kernel,
    mesh=_mesh,
    compiler_params=pltpu.CompilerParams(needs_layout_passes=False),
    out_type=jax.ShapeDtypeStruct((_SC_ROWS, _N), jnp.int32),
    scratch_types=[
        pltpu.VMEM((_CHUNK,), jnp.float32),
        pltpu.VMEM((_CHUNK,), jnp.float32),
        pltpu.VMEM((_N,), jnp.int32),
        pltpu.VMEM((2048,), jnp.int32),
        pltpu.VMEM((_N,), jnp.int32),
        pltpu.VMEM((_L,), jnp.int32),
    ],
)
def _sc_mask(p_hbm, u_hbm, k_hbm, out_hbm, p_buf, u_buf, key_buf, hist, out_buf, kvec):
    row = lax.axis_index("s") * _NC + lax.axis_index("c")
    pltpu.sync_copy(k_hbm, kvec)
    k = jnp.sum(kvec[...])
    ones = jnp.ones((_L,), jnp.int32)

    _zero_hist(hist, 2048)

    def chunk_body(c, _):
        pltpu.sync_copy(p_hbm.at[row, pl.ds(c * _CHUNK, _CHUNK)], p_buf)
        pltpu.sync_copy(u_hbm.at[row, pl.ds(c * _CHUNK, _CHUNK)], u_buf)

        def kg_body(j, _):
            for i in range(_U):
                off = (j * _U + i) * _L
                pv = p_buf[pl.ds(off, _L)]
                uv = u_buf[pl.ds(off, _L)]
                kb = _keys_for(pv, uv)
                key_buf[pl.ds(c * _CHUNK + off, _L)] = kb
                plsc.addupdate_scatter(hist, [kb >> 20], ones)
            return 0

        lax.fori_loop(0, _CHUNK // (_L * _U), kg_body, 0)
        return 0

    lax.fori_loop(0, _NCHUNKS, chunk_body, 0)

    need = k
    D1, cb1 = _digit_select(hist, 2048, need)
    pfx = D1
    need = need - cb1

    _zero_hist(hist, 1024)

    def h2_body(j, _):
        for i in range(_U):
            kv = key_buf[pl.ds((j * _U + i) * _L, _L)]
            match = (kv >> 20) == pfx
            plsc.addupdate_scatter(hist, [(kv >> 10) & 1023], ones, mask=match)
        return 0

    lax.fori_loop(0, _N // (_L * _U), h2_body, 0)
    D2, cb2 = _digit_select(hist, 1024, need)
    pfx = (pfx << 10) | D2
    need = need - cb2

    _zero_hist(hist, 1024)

    def h3_body(j, _):
        for i in range(_U):
            kv = key_buf[pl.ds((j * _U + i) * _L, _L)]
            match = (kv >> 10) == pfx
            plsc.addupdate_scatter(hist, [kv & 1023], ones, mask=match)
        return 0

    lax.fori_loop(0, _N // (_L * _U), h3_body, 0)
    D3, cb3 = _digit_select(hist, 1024, need)
    T = (pfx << 10) | D3
    need = need - cb3

    def out_body(j, cnt):
        for i in range(_U):
            off = (j * _U + i) * _L
            kv = key_buf[pl.ds(off, _L)]
            ltm = kv < T
            eqm = kv == T
            eqi = eqm.astype(jnp.int32)
            rank = plsc.cumsum(eqi) + cnt
            sel = eqm & (rank <= need)
            out_buf[pl.ds(off, _L)] = (ltm | sel).astype(jnp.int32)
            cnt = cnt + jnp.sum(eqi)
        return cnt

    lax.fori_loop(0, _N // (_L * _U), out_body, jnp.int32(0))
    pltpu.sync_copy(out_buf, out_hbm.at[row])


def _tc_mask_kernel(k_ref, probs_ref, gumbel_ref, out_ref):
    k = k_ref[0]
    p = probs_ref[...]
    u = gumbel_ref[...]

    eps = 1e-20
    inner = -jnp.log(jnp.maximum(u, eps))
    r = (p + 1e-05) / inner
    ukey = lax.bitcast_convert_type(r, jnp.uint32)

    rows = p.shape[0]
    ones_bf = jnp.ones((_N, 1), jnp.bfloat16)
    one_b = jnp.bfloat16(1)
    zero_b = jnp.bfloat16(0)
    k_f = k.astype(jnp.float32)

    bias = jnp.uint32(0x8000)
    hi = (ukey >> 16).astype(jnp.int16)
    lo = (ukey ^ bias).astype(jnp.int16)

    def count_lt(arr, cand_i16):
        sel = jnp.where(arr < cand_i16, one_b, zero_b)
        return jax.lax.dot_general(
            sel, ones_bf, (((1,), (0,)), ((), ())),
            preferred_element_type=jnp.float32,
        )

    def hi_body(_, carry):
        tpref, bit = carry
        cand = tpref | bit
        cnt = count_lt(hi, cand.astype(jnp.int16))
        tpref = jnp.where(cnt >= k_f, tpref, cand)
        return tpref, bit >> 1

    t0 = jnp.zeros((rows, 1), jnp.int32)
    T_hi, _ = lax.fori_loop(0, 15, hi_body, (t0, jnp.int32(1 << 14)))

    t_hi_i16 = T_hi.astype(jnp.int16)
    c_hi = count_lt(hi, t_hi_i16)
    need_lo = k_f - c_hi
    pm = hi == t_hi_i16
    masked_lo = jnp.where(pm, lo, jnp.int16(32767))

    sb = jnp.int32(0x8000)

    def lo_body(_, carry):
        tpref, bit = carry
        cand = tpref | bit
        cnt = count_lt(masked_lo, (cand ^ sb).astype(jnp.int16))
        tpref = jnp.where(cnt >= need_lo, tpref, cand)
        return tpref, bit >> 1

    T_lo, _ = lax.fori_loop(0, 16, lo_body, (t0, jnp.int32(1 << 15)))

    T = (T_hi.astype(jnp.uint32) << 16) | T_lo.astype(jnp.uint32)
    lt = ukey < T
    eq = ukey == T
    ltf = jnp.where(lt, 1.0, 0.0)
    c_lt = jax.lax.dot_general(
        ltf, jnp.ones((_N, 1), jnp.float32), (((1,), (0,)), ((), ())),
        preferred_element_type=jnp.float32,
    )
    need = k_f - c_lt

    eqf = jnp.where(eq, 1.0, 0.0)
    e2 = eqf.reshape(rows * _CHUNKS, _LANES)
    li = lax.broadcasted_iota(jnp.int32, (_LANES, _LANES), 0)
    lj = lax.broadcasted_iota(jnp.int32, (_LANES, _LANES), 1)
    lt_incl = jnp.where(li <= lj, 1.0, 0.0)
    intra = jax.lax.dot_general(
        e2, lt_incl, (((1,), (0,)), ((), ())),
        preferred_element_type=jnp.float32,
    )
    totals = jax.lax.dot_general(
        e2, jnp.ones((_LANES, 1), jnp.float32), (((1,), (0,)), ((), ())),
        preferred_element_type=jnp.float32,
    ).reshape(rows, _CHUNKS)
    ci = lax.broadcasted_iota(jnp.int32, (_CHUNKS, _CHUNKS), 0)
    cj = lax.broadcasted_iota(jnp.int32, (_CHUNKS, _CHUNKS), 1)
    slt = jnp.where(ci < cj, 1.0, 0.0)
    offs = jax.lax.dot_general(
        totals, slt, (((1,), (0,)), ((), ())),
        preferred_element_type=jnp.float32,
    )
    rank = intra.reshape(rows, _CHUNKS, _LANES) + offs[:, :, None]
    rank = rank.reshape(rows, _N)

    out_ref[...] = lt | (eq & (rank <= need))


def _tc_mask(probs, gumbel_u, k):
    skip = _SC_ROWS // _TC_BLOCK
    grid = (_TC_ROWS // _TC_BLOCK,)
    return pl.pallas_call(
        _tc_mask_kernel,
        grid=grid,
        in_specs=[
            pl.BlockSpec(memory_space=pltpu.SMEM),
            pl.BlockSpec((_TC_BLOCK, _N), lambda i: (i + skip, 0)),
            pl.BlockSpec((_TC_BLOCK, _N), lambda i: (i + skip, 0)),
        ],
        out_specs=pl.BlockSpec((_TC_BLOCK, _N), lambda i: (i, 0)),
        out_shape=jax.ShapeDtypeStruct((_TC_ROWS, _N), jnp.bool_),
    )(k, probs, gumbel_u)


def kernel(probs, gumbel_u, mask_len):
    k32 = jnp.asarray(mask_len, jnp.int32)
    karr = jnp.zeros((_L,), jnp.int32).at[0].set(k32)
    sc_out = _sc_mask(probs, gumbel_u, karr)
    tc_out = _tc_mask(probs, gumbel_u, k32.reshape(1))
    return jnp.concatenate([sc_out.astype(jnp.bool_), tc_out], axis=0)


# speedup vs baseline: 1.8072x; 1.8072x over previous
import functools

import jax
import jax.numpy as jnp
from jax import lax
from jax.experimental import pallas as pl
from jax.experimental.pallas import tpu as pltpu
from jax.experimental.pallas import tpu_sc as plsc

_N = 32768
_LANES = 128
_CHUNKS = _N // _LANES


_L = 16
_NC = 2
_NS = 16
_NW = _NC * _NS
_SC_ROWS = 32
_TC_ROWS = 96
_TC_BLOCK = 48
_CHUNK = 8192
_NCHUNKS = _N // _CHUNK
_U = 8

_LN2 = 0.6931471805599453
_SQRT2 = 1.4142135623730951
_LOG2_COEF = (
    2.8978064198215137e-08,
    1.4426949525633355,
    -0.7213581907292554,
    0.4809198929627641,
    -0.3600796321545737,
    0.28720812481307534,
    -0.2504655854924836,
    0.23321562654833375,
    -0.14022713339628334,
)

_mesh = plsc.VectorSubcoreMesh(core_axis_name="c", subcore_axis_name="s")


def _keys_for(pv, uv):
    ub = lax.bitcast_convert_type(uv, jnp.int32)
    e = (ub >> 23) - 127
    m = lax.bitcast_convert_type((ub & 0x7FFFFF) | 0x3F800000, jnp.float32)
    adj = m > jnp.float32(_SQRT2)
    m = jnp.where(adj, m * jnp.float32(0.5), m)
    e = e + adj.astype(jnp.int32)
    t = m - jnp.float32(1.0)
    acc = jnp.full((_L,), _LOG2_COEF[-1], jnp.float32)
    for c in _LOG2_COEF[-2::-1]:
        acc = acc * t + jnp.float32(c)
    logu = (e.astype(jnp.float32) + acc) * jnp.float32(_LN2)
    inner = jnp.maximum(-logu, jnp.float32(1e-9))
    r = (pv + jnp.float32(1e-5)) / inner
    return lax.bitcast_convert_type(r, jnp.int32)


def _digit_select(hist_ref, nbins, need):
    def body(v, carry):
        run, dcnt, cbel = carry
        h = hist_ref[pl.ds(v * _L, _L)]
        s = plsc.cumsum(h) + run
        mlt = s < need
        dcnt = dcnt + jnp.sum(mlt.astype(jnp.int32))
        cbel = cbel + jnp.sum(jnp.where(mlt, h, 0))
        run = run + jnp.sum(h)
        return run, dcnt, cbel

    z = jnp.int32(0)
    _, D, c_below = lax.fori_loop(0, nbins // _L, body, (z, z, z))
    return D, c_below


def _zero_hist(hist_ref, nbins):
    def body(v, _):
        hist_ref[pl.ds(v * _L, _L)] = jnp.zeros((_L,), jnp.int32)
        return 0

    lax.fori_loop(0, nbins // _L, body, 0)


@functools.partial(
    pl.kernel,
    mesh=_mesh,
    compiler_params=pltpu.CompilerParams(needs_layout_passes=False),
    out_type=jax.ShapeDtypeStruct((_SC_ROWS, _N), jnp.int32),
    scratch_types=[
        pltpu.VMEM((_CHUNK,), jnp.float32),
        pltpu.VMEM((_CHUNK,), jnp.float32),
        pltpu.VMEM((_N,), jnp.int32),
        pltpu.VMEM((2048,), jnp.int32),
        pltpu.VMEM((_N,), jnp.int32),
        pltpu.VMEM((_L,), jnp.int32),
    ],
)
def _sc_mask(p_hbm, u_hbm, k_hbm, out_hbm, p_buf, u_buf, key_buf, hist, out_buf, kvec):
    row = lax.axis_index("s") * _NC + lax.axis_index("c")
    in_row = _TC_ROWS + row
    pltpu.sync_copy(k_hbm, kvec)
    k = jnp.sum(kvec[...])
    ones = jnp.ones((_L,), jnp.int32)

    _zero_hist(hist, 2048)

    def chunk_body(c, _):
        pltpu.sync_copy(p_hbm.at[in_row, pl.ds(c * _CHUNK, _CHUNK)], p_buf)
        pltpu.sync_copy(u_hbm.at[in_row, pl.ds(c * _CHUNK, _CHUNK)], u_buf)

        def kg_body(j, _):
            for i in range(_U):
                off = (j * _U + i) * _L
                pv = p_buf[pl.ds(off, _L)]
                uv = u_buf[pl.ds(off, _L)]
                kb = _keys_for(pv, uv)
                key_buf[pl.ds(c * _CHUNK + off, _L)] = kb
                plsc.addupdate_scatter(hist, [kb >> 20], ones)
            return 0

        lax.fori_loop(0, _CHUNK // (_L * _U), kg_body, 0)
        return 0

    lax.fori_loop(0, _NCHUNKS, chunk_body, 0)

    need = k
    D1, cb1 = _digit_select(hist, 2048, need)
    pfx = D1
    need = need - cb1

    _zero_hist(hist, 1024)

    def h2_body(j, _):
        for i in range(_U):
            kv = key_buf[pl.ds((j * _U + i) * _L, _L)]
            match = (kv >> 20) == pfx
            plsc.addupdate_scatter(hist, [(kv >> 10) & 1023], ones, mask=match)
        return 0

    lax.fori_loop(0, _N // (_L * _U), h2_body, 0)
    D2, cb2 = _digit_select(hist, 1024, need)
    pfx = (pfx << 10) | D2
    need = need - cb2

    _zero_hist(hist, 1024)

    def h3_body(j, _):
        for i in range(_U):
            kv = key_buf[pl.ds((j * _U + i) * _L, _L)]
            match = (kv >> 10) == pfx
            plsc.addupdate_scatter(hist, [kv & 1023], ones, mask=match)
        return 0

    lax.fori_loop(0, _N // (_L * _U), h3_body, 0)
    D3, cb3 = _digit_select(hist, 1024, need)
    T = (pfx << 10) | D3
    need = need - cb3

    def out_body(j, cnt):
        for i in range(_U):
            off = (j * _U + i) * _L
            kv = key_buf[pl.ds(off, _L)]
            ltm = kv < T
            eqm = kv == T
            eqi = eqm.astype(jnp.int32)
            rank = plsc.cumsum(eqi) + cnt
            sel = eqm & (rank <= need)
            out_buf[pl.ds(off, _L)] = (ltm | sel).astype(jnp.int32)
            cnt = cnt + jnp.sum(eqi)
        return cnt

    lax.fori_loop(0, _N // (_L * _U), out_body, jnp.int32(0))
    pltpu.sync_copy(out_buf, out_hbm.at[row])


def _tc_mask_kernel(k_ref, probs_ref, gumbel_ref, out_ref):
    k = k_ref[0]
    p = probs_ref[...]
    u = gumbel_ref[...]

    eps = 1e-20
    inner = -jnp.log(jnp.maximum(u, eps))
    r = (p + 1e-05) / inner
    ukey = lax.bitcast_convert_type(r, jnp.uint32)

    rows = p.shape[0]
    ones_bf = jnp.ones((_N, 1), jnp.bfloat16)
    one_b = jnp.bfloat16(1)
    zero_b = jnp.bfloat16(0)
    k_f = k.astype(jnp.float32)

    bias = jnp.uint32(0x8000)
    hi = (ukey >> 16).astype(jnp.int16)
    lo = (ukey ^ bias).astype(jnp.int16)

    def count_lt(arr, cand_i16):
        sel = jnp.where(arr < cand_i16, one_b, zero_b)
        return jax.lax.dot_general(
            sel, ones_bf, (((1,), (0,)), ((), ())),
            preferred_element_type=jnp.float32,
        )

    def hi_body(_, carry):
        tpref, bit = carry
        cand = tpref | bit
        cnt = count_lt(hi, cand.astype(jnp.int16))
        tpref = jnp.where(cnt >= k_f, tpref, cand)
        return tpref, bit >> 1

    t0 = jnp.zeros((rows, 1), jnp.int32)
    T_hi, _ = lax.fori_loop(0, 15, hi_body, (t0, jnp.int32(1 << 14)))

    t_hi_i16 = T_hi.astype(jnp.int16)
    c_hi = count_lt(hi, t_hi_i16)
    need_lo = k_f - c_hi
    pm = hi == t_hi_i16
    masked_lo = jnp.where(pm, lo, jnp.int16(32767))

    sb = jnp.int32(0x8000)

    def lo_body(_, carry):
        tpref, bit = carry
        cand = tpref | bit
        cnt = count_lt(masked_lo, (cand ^ sb).astype(jnp.int16))
        tpref = jnp.where(cnt >= need_lo, tpref, cand)
        return tpref, bit >> 1

    T_lo, _ = lax.fori_loop(0, 16, lo_body, (t0, jnp.int32(1 << 15)))

    T = (T_hi.astype(jnp.uint32) << 16) | T_lo.astype(jnp.uint32)
    lt = ukey < T
    eq = ukey == T
    ltf = jnp.where(lt, 1.0, 0.0)
    c_lt = jax.lax.dot_general(
        ltf, jnp.ones((_N, 1), jnp.float32), (((1,), (0,)), ((), ())),
        preferred_element_type=jnp.float32,
    )
    need = k_f - c_lt

    eqf = jnp.where(eq, 1.0, 0.0)
    e2 = eqf.reshape(rows * _CHUNKS, _LANES)
    li = lax.broadcasted_iota(jnp.int32, (_LANES, _LANES), 0)
    lj = lax.broadcasted_iota(jnp.int32, (_LANES, _LANES), 1)
    lt_incl = jnp.where(li <= lj, 1.0, 0.0)
    intra = jax.lax.dot_general(
        e2, lt_incl, (((1,), (0,)), ((), ())),
        preferred_element_type=jnp.float32,
    )
    totals = jax.lax.dot_general(
        e2, jnp.ones((_LANES, 1), jnp.float32), (((1,), (0,)), ((), ())),
        preferred_element_type=jnp.float32,
    ).reshape(rows, _CHUNKS)
    ci = lax.broadcasted_iota(jnp.int32, (_CHUNKS, _CHUNKS), 0)
    cj = lax.broadcasted_iota(jnp.int32, (_CHUNKS, _CHUNKS), 1)
    slt = jnp.where(ci < cj, 1.0, 0.0)
    offs = jax.lax.dot_general(
        totals, slt, (((1,), (0,)), ((), ())),
        preferred_element_type=jnp.float32,
    )
    rank = intra.reshape(rows, _CHUNKS, _LANES) + offs[:, :, None]
    rank = rank.reshape(rows, _N)

    out_ref[...] = lt | (eq & (rank <= need))


def _tc_mask(probs, gumbel_u, k):
    grid = (_TC_ROWS // _TC_BLOCK,)
    return pl.pallas_call(
        _tc_mask_kernel,
        grid=grid,
        in_specs=[
            pl.BlockSpec(memory_space=pltpu.SMEM),
            pl.BlockSpec((_TC_BLOCK, _N), lambda i: (i, 0)),
            pl.BlockSpec((_TC_BLOCK, _N), lambda i: (i, 0)),
        ],
        out_specs=pl.BlockSpec((_TC_BLOCK, _N), lambda i: (i, 0)),
        out_shape=jax.ShapeDtypeStruct((_TC_ROWS, _N), jnp.bool_),
    )(k, probs, gumbel_u)


def kernel(probs, gumbel_u, mask_len):
    k32 = jnp.asarray(mask_len, jnp.int32)
    karr = jnp.zeros((_L,), jnp.int32).at[0].set(k32)
    sc_out = _sc_mask(probs, gumbel_u, karr)
    tc_out = _tc_mask(probs, gumbel_u, k32.reshape(1))
    return jnp.concatenate([tc_out, sc_out.astype(jnp.bool_)], axis=0)
